# baseline (speedup 1.0000x reference)
import functools

import jax
import jax.numpy as jnp
from jax import lax
from jax.experimental import pallas as pl
from jax.experimental.pallas import tpu as pltpu
from jax.experimental.pallas import tpu_sc as plsc

BATCH = 16384
EMBED_DIM = 32

_NC = 2
_NS = 16
_NW = _NC * _NS
_B_PER_W = BATCH // _NW
_CHUNK = 128
_NCHUNK = _B_PER_W // _CHUNK


def _sc_gather_body(uids_hbm, iids_hbm, utab_hbm, itab_hbm,
                    u_out, i_out,
                    uidx_v, iidx_v, urows_v, irows_v, sem):
    wid = lax.axis_index("s") * _NC + lax.axis_index("c")
    base = wid * _B_PER_W
    pltpu.sync_copy(uids_hbm.at[pl.ds(base, _B_PER_W)], uidx_v)
    pltpu.sync_copy(iids_hbm.at[pl.ds(base, _B_PER_W)], iidx_v)
    copies = []
    for c in range(_NCHUNK):
        sl = pl.ds(c * _CHUNK, _CHUNK)
        copies.append(
            pltpu.async_copy(utab_hbm.at[uidx_v.at[sl]], urows_v.at[sl], sem))
        copies.append(
            pltpu.async_copy(itab_hbm.at[iidx_v.at[sl]], irows_v.at[sl], sem))
    for cp in copies:
        cp.wait()
    pltpu.sync_copy(urows_v, u_out.at[pl.ds(base, _B_PER_W)])
    pltpu.sync_copy(irows_v, i_out.at[pl.ds(base, _B_PER_W)])


def _sc_gather(user_ids, item_ids, user_table, item_table):
    mesh = plsc.VectorSubcoreMesh(
        core_axis_name="c", subcore_axis_name="s",
        num_cores=_NC, num_subcores=_NS)
    f = pl.kernel(
        _sc_gather_body,
        out_type=(
            jax.ShapeDtypeStruct((BATCH, EMBED_DIM), jnp.float32),
            jax.ShapeDtypeStruct((BATCH, EMBED_DIM), jnp.float32),
        ),
        mesh=mesh,
        scratch_types=(
            pltpu.VMEM((_B_PER_W,), jnp.int32),
            pltpu.VMEM((_B_PER_W,), jnp.int32),
            pltpu.VMEM((_B_PER_W, EMBED_DIM), jnp.float32),
            pltpu.VMEM((_B_PER_W, EMBED_DIM), jnp.float32),
            pltpu.SemaphoreType.DMA,
        ),
        compiler_params=pltpu.CompilerParams(use_tc_tiling_on_sc=False),
    )
    return f(user_ids, item_ids, user_table, item_table)


_MLP_BLK = 2048


def _mlp_body(u_ref, i_ref, w1a_ref, w1b_ref, b1_ref, w2_ref, b2_ref,
              w3_ref, b3_ref, w4_ref, b4_ref, out_ref):
    f32 = jnp.float32
    h = (jnp.dot(u_ref[...], w1a_ref[...], preferred_element_type=f32)
         + jnp.dot(i_ref[...], w1b_ref[...], preferred_element_type=f32)
         + b1_ref[...])
    h = jnp.maximum(h, 0.0)
    h = jnp.dot(h, w2_ref[...], preferred_element_type=f32) + b2_ref[...]
    h = jnp.maximum(h, 0.0)
    h = jnp.dot(h, w3_ref[...], preferred_element_type=f32) + b3_ref[...]
    h = jnp.maximum(h, 0.0)
    z = jnp.dot(h, w4_ref[...], preferred_element_type=f32) + b4_ref[...]
    out_ref[...] = jax.nn.sigmoid(z)


def _mlp(u, i, W1, b1, W2, b2, W3, b3, W4, b4):
    w1a = W1[:EMBED_DIM]
    w1b = W1[EMBED_DIM:]
    grid = BATCH // _MLP_BLK
    full = lambda a: pl.BlockSpec(a.shape, lambda b: (0,) * a.ndim)
    out = pl.pallas_call(
        _mlp_body,
        grid=(grid,),
        in_specs=[
            pl.BlockSpec((_MLP_BLK, EMBED_DIM), lambda b: (b, 0)),
            pl.BlockSpec((_MLP_BLK, EMBED_DIM), lambda b: (b, 0)),
            full(w1a), full(w1b),
            pl.BlockSpec((1, 64), lambda b: (0, 0)),
            full(W2),
            pl.BlockSpec((1, 32), lambda b: (0, 0)),
            full(W3),
            pl.BlockSpec((1, 16), lambda b: (0, 0)),
            full(W4),
            pl.BlockSpec((1, 1), lambda b: (0, 0)),
        ],
        out_specs=pl.BlockSpec((_MLP_BLK, 1), lambda b: (b, 0)),
        out_shape=jax.ShapeDtypeStruct((BATCH, 1), jnp.float32),
    )(u, i, w1a, w1b, b1.reshape(1, 64), W2, b2.reshape(1, 32),
      W3, b3.reshape(1, 16), W4, b4.reshape(1, 1))
    return out[:, 0]


def kernel(user_ids, item_ids, user_table, item_table,
           W1, b1, W2, b2, W3, b3, W4, b4):
    u, i = _sc_gather(user_ids, item_ids, user_table, item_table)
    return _mlp(u, i, W1, b1, W2, b2, W3, b3, W4, b4)
